# baseline (speedup 1.0000x reference)
import functools

import jax
import jax.numpy as jnp
from jax import lax
from jax.experimental import pallas as pl
from jax.experimental.pallas import tpu as pltpu
from jax.experimental.pallas import tpu_sc as plsc

_B = 4096
_L = 50
_NEG = 5
_E = 32
_LN = _L * _NEG
_NW = 32
_BW = _B // _NW
_W = 5
_NG = 2 * (_L + _LN) // _W

_mesh = plsc.VectorSubcoreMesh(core_axis_name="c", subcore_axis_name="s")


@functools.partial(
    pl.kernel,
    out_type=jax.ShapeDtypeStruct((_B, 7 * _E), jnp.float32),
    mesh=_mesh,
    scratch_types=[
        pltpu.VMEM((3, _BW), jnp.int32),
        pltpu.VMEM((2, _W, _BW), jnp.int32),
        pltpu.VMEM((2, _W, _BW, _E), jnp.float32),
        pltpu.VMEM((3, _BW, _E), jnp.float32),
        pltpu.VMEM((_BW, 7 * _E), jnp.float32),
        pltpu.SemaphoreType.DMA,
        pltpu.SemaphoreType.DMA,
        pltpu.SemaphoreType.DMA,
    ],
    compiler_params=pltpu.CompilerParams(use_tc_tiling_on_sc=False),
)
def _embed_kernel(bigT, uid_hbm, mid_hbm, cat_hbm, uid_table, mid_table,
                  cat_table, out_hbm, sidx, idx_v, dbuf, sbuf, stage,
                  sem0, sem1, sem_s):
    wid = lax.axis_index("s") * 2 + lax.axis_index("c")
    base = wid * _BW
    sems = (sem0, sem1)

    pltpu.sync_copy(uid_hbm.at[pl.ds(base, _BW)], sidx.at[0])
    pltpu.sync_copy(mid_hbm.at[pl.ds(base, _BW)], sidx.at[1])
    pltpu.sync_copy(cat_hbm.at[pl.ds(base, _BW)], sidx.at[2])
    pltpu.async_copy(uid_table.at[sidx.at[0]], sbuf.at[0], sem_s)
    pltpu.async_copy(mid_table.at[sidx.at[1]], sbuf.at[1], sem_s)
    pltpu.async_copy(cat_table.at[sidx.at[2]], sbuf.at[2], sem_s)

    def make_fetch(table):
        def fetch(g, slot):
            pltpu.sync_copy(bigT.at[pl.ds(g * _W, _W), pl.ds(base, _BW)],
                            idx_v.at[slot])
            for k in range(_W):
                pltpu.async_copy(table.at[idx_v.at[slot, k]],
                                 dbuf.at[slot, k], sems[slot])
        return fetch

    def consume(g, slot, db):
        for k in range(_W):
            pltpu.make_async_copy(mid_table.at[idx_v.at[slot, k]],
                                  dbuf.at[slot, k], sems[slot]).wait()
        dv = dbuf.at[slot]

        def row(r, carry):
            for h in range(2):
                sl = pl.ds(h * 16, 16)
                acc = stage[r, pl.ds(db + h * 16, 16)]
                for k in range(_W):
                    acc = acc + dv[k, r, sl]
                stage[r, pl.ds(db + h * 16, 16)] = acc
            return carry

        lax.fori_loop(0, _BW, row, 0, unroll=4)

    z = jnp.zeros((16,), jnp.float32)

    def zrow(r, carry):
        for v in range(8):
            stage[r, pl.ds(3 * _E + v * 16, 16)] = z
        return carry

    lax.fori_loop(0, _BW, zrow, 0, unroll=4)

    for t in range(3):
        pltpu.make_async_copy(uid_table.at[sidx.at[0]], sbuf.at[t],
                              sem_s).wait()

    def srow(r, carry):
        for t in range(3):
            for h in range(2):
                stage[r, pl.ds(t * _E + h * 16, 16)] = sbuf[t, r,
                                                            pl.ds(h * 16, 16)]
        return carry

    lax.fori_loop(0, _BW, srow, 0, unroll=4)

    fetch_mid = make_fetch(mid_table)
    fetch_cat = make_fetch(cat_table)

    def run_phase(fetch, lo, hi, split, db_lo, db_hi):
        fetch(lo, 0)

        def step(i, carry):
            for par in range(2):
                g = 2 * i + par
                fetch(jnp.minimum(g + 1, hi - 1), 1 - par)
                db = jnp.where(g < split, db_lo, db_hi)
                consume(g, par, db)
            return carry

        lax.fori_loop(lo // 2, hi // 2, step, 0)
        for k in range(_W):
            pltpu.make_async_copy(mid_table.at[idx_v.at[0, k]],
                                  dbuf.at[0, k], sem0).wait()

    run_phase(fetch_mid, 0, 60, 10, 3 * _E, 5 * _E)
    run_phase(fetch_cat, 60, 120, 70, 4 * _E, 6 * _E)

    pltpu.sync_copy(stage, out_hbm.at[pl.ds(base, _BW)])


def kernel(uid_batch, mid_batch, cat_batch, mid_his_batch, cat_his_batch,
           noclk_mid_batch, noclk_cat_batch, uid_table, mid_table, cat_table):
    bigT = jnp.concatenate([
        mid_his_batch.T,
        noclk_mid_batch.transpose(2, 1, 0).reshape(_LN, _B),
        cat_his_batch.T,
        noclk_cat_batch.transpose(2, 1, 0).reshape(_LN, _B),
    ], axis=0)
    return _embed_kernel(bigT, uid_batch, mid_batch, cat_batch,
                         uid_table, mid_table, cat_table)


# speedup vs baseline: 1.1415x; 1.1415x over previous
import functools

import jax
import jax.numpy as jnp
from jax import lax
from jax.experimental import pallas as pl
from jax.experimental.pallas import tpu as pltpu
from jax.experimental.pallas import tpu_sc as plsc

_B = 4096
_L = 50
_NEG = 5
_E = 32
_P = _L + _L * _NEG
_NW = 32
_BW = _B // _NW
_R = 2
_RP = _R * _P
_G = _BW // _R

_mesh = plsc.VectorSubcoreMesh(core_axis_name="c", subcore_axis_name="s")


@functools.partial(
    pl.kernel,
    out_type=jax.ShapeDtypeStruct((_B, 7 * _E), jnp.float32),
    mesh=_mesh,
    scratch_types=[
        pltpu.VMEM((3, _BW), jnp.int32),
        pltpu.VMEM((3, _BW, _E), jnp.float32),
        pltpu.VMEM((2, 2, _RP), jnp.int32),
        pltpu.VMEM((2, _RP, _E), jnp.float32),
        pltpu.VMEM((2, _RP, _E), jnp.float32),
        pltpu.VMEM((_BW, 7 * _E), jnp.float32),
        pltpu.SemaphoreType.DMA,
        pltpu.SemaphoreType.DMA,
        pltpu.SemaphoreType.DMA,
        pltpu.SemaphoreType.DMA,
    ],
    compiler_params=pltpu.CompilerParams(use_tc_tiling_on_sc=False),
)
def _embed_kernel(idx_hbm, uid_hbm, mid_hbm, cat_hbm, uid_table, mid_table,
                  cat_table, out_hbm, sidx, sbuf, idx_v, mbuf, cbuf, stage,
                  sem_m0, sem_m1, sem_c0, sem_c1):
    wid = lax.axis_index("s") * 2 + lax.axis_index("c")
    base = wid * _BW
    chunk0 = wid * _G
    sem_m = (sem_m0, sem_m1)
    sem_c = (sem_c0, sem_c1)

    pltpu.sync_copy(uid_hbm.at[pl.ds(base, _BW)], sidx.at[0])
    pltpu.sync_copy(mid_hbm.at[pl.ds(base, _BW)], sidx.at[1])
    pltpu.sync_copy(cat_hbm.at[pl.ds(base, _BW)], sidx.at[2])
    pltpu.async_copy(uid_table.at[sidx.at[0]], sbuf.at[0], sem_m0)
    pltpu.async_copy(mid_table.at[sidx.at[1]], sbuf.at[1], sem_m0)
    pltpu.async_copy(cat_table.at[sidx.at[2]], sbuf.at[2], sem_m0)

    def fetch(g, slot):
        pltpu.sync_copy(idx_hbm.at[chunk0 + g], idx_v.at[slot])
        pltpu.async_copy(mid_table.at[idx_v.at[slot, 0]], mbuf.at[slot],
                         sem_m[slot])
        pltpu.async_copy(cat_table.at[idx_v.at[slot, 1]], cbuf.at[slot],
                         sem_c[slot])

    def consume(g, slot):
        pltpu.make_async_copy(mid_table.at[idx_v.at[slot, 0]], mbuf.at[slot],
                              sem_m[slot]).wait()
        pltpu.make_async_copy(cat_table.at[idx_v.at[slot, 1]], cbuf.at[slot],
                              sem_c[slot]).wait()
        mb = mbuf.at[slot]
        cb = cbuf.at[slot]
        for r in range(_R):
            p = r * _P
            row = g * _R + r
            for h in range(2):
                sl = pl.ds(h * 16, 16)
                for t in range(3):
                    stage[row, pl.ds(t * _E + h * 16, 16)] = sbuf[t, row, sl]

            def seg(lo, hi, unroll):
                z = jnp.zeros((16,), jnp.float32)

                def body(j, c):
                    return (c[0] + mb[p + j, pl.ds(0, 16)],
                            c[1] + mb[p + j, pl.ds(16, 16)],
                            c[2] + cb[p + j, pl.ds(0, 16)],
                            c[3] + cb[p + j, pl.ds(16, 16)])

                return lax.fori_loop(lo, hi, body, (z, z, z, z),
                                     unroll=unroll)

            m0, m1, c0, c1 = seg(0, _L, 10)
            stage[row, pl.ds(3 * _E, 16)] = m0
            stage[row, pl.ds(3 * _E + 16, 16)] = m1
            stage[row, pl.ds(4 * _E, 16)] = c0
            stage[row, pl.ds(4 * _E + 16, 16)] = c1
            m0, m1, c0, c1 = seg(_L, _P, 25)
            stage[row, pl.ds(5 * _E, 16)] = m0
            stage[row, pl.ds(5 * _E + 16, 16)] = m1
            stage[row, pl.ds(6 * _E, 16)] = c0
            stage[row, pl.ds(6 * _E + 16, 16)] = c1

    for t in range(3):
        pltpu.make_async_copy(uid_table.at[sidx.at[0]], sbuf.at[t],
                              sem_m0).wait()
    fetch(0, 0)

    def step(i, carry):
        for par in range(2):
            g = 2 * i + par
            gn = jnp.minimum(g + 1, _G - 1)
            fetch(gn, 1 - par)
            consume(g, par)
        return carry

    lax.fori_loop(0, _G // 2, step, 0)
    pltpu.make_async_copy(mid_table.at[idx_v.at[0, 0]], mbuf.at[0],
                          sem_m0).wait()
    pltpu.make_async_copy(cat_table.at[idx_v.at[0, 1]], cbuf.at[0],
                          sem_c0).wait()

    pltpu.sync_copy(stage, out_hbm.at[pl.ds(base, _BW)])


def kernel(uid_batch, mid_batch, cat_batch, mid_his_batch, cat_his_batch,
           noclk_mid_batch, noclk_cat_batch, uid_table, mid_table, cat_table):
    midx = jnp.concatenate(
        [mid_his_batch, noclk_mid_batch.reshape(_B, _L * _NEG)], axis=1)
    catx = jnp.concatenate(
        [cat_his_batch, noclk_cat_batch.reshape(_B, _L * _NEG)], axis=1)
    idx = jnp.stack([midx.reshape(_B // _R, _RP),
                     catx.reshape(_B // _R, _RP)], axis=1)
    return _embed_kernel(idx, uid_batch, mid_batch, cat_batch,
                         uid_table, mid_table, cat_table)
